# baseline (speedup 1.0000x reference)
import functools

import jax
import jax.numpy as jnp
from jax import lax
from jax.experimental import pallas as pl
from jax.experimental.pallas import tpu as pltpu
from jax.experimental.pallas import tpu_sc as plsc

NC = 2
NS = 16
NW = NC * NS
LANES = 16
Q = 4


def kernel(user, item, user_table, item_table):
    B = user.shape[0]
    D = user_table.shape[1]
    BPW = B // NW

    ut = user_table.T
    it = item_table.T

    mesh = plsc.VectorSubcoreMesh(core_axis_name="c", subcore_axis_name="s")

    @functools.partial(
        pl.kernel,
        out_type=jax.ShapeDtypeStruct((B,), jnp.float32),
        mesh=mesh,
        scratch_types=[
            pltpu.VMEM((BPW,), jnp.int32),
            pltpu.VMEM((BPW,), jnp.int32),
            pltpu.VMEM((Q, D, 128), jnp.float32),
            pltpu.VMEM((Q, D, 128), jnp.float32),
            pltpu.VMEM((BPW,), jnp.float32),
            pltpu.SemaphoreType.DMA((Q,)),
            pltpu.SemaphoreType.DMA((Q,)),
        ],
        compiler_params=pltpu.CompilerParams(
            needs_layout_passes=False, use_tc_tiling_on_sc=True),
    )
    def _emb_dot(uidx_hbm, iidx_hbm, ut_hbm, it_hbm, out_hbm,
                 uidx_v, iidx_v, u_pan, i_pan, out_v, sem_u, sem_i):
        wid = lax.axis_index("s") * NC + lax.axis_index("c")
        base = wid * BPW

        pltpu.sync_copy(uidx_hbm.at[pl.ds(base, BPW)], uidx_v)
        pltpu.sync_copy(iidx_hbm.at[pl.ds(base, BPW)], iidx_v)

        iota = lax.iota(jnp.int32, LANES)
        zero = jnp.zeros((LANES,), jnp.int32)

        def extract(vref, k):
            vec = vref[pl.ds((k // LANES) * LANES, LANES)]
            return jnp.sum(jnp.where(iota == (k % LANES), vec, zero))

        H = D // 2

        def issue(k, q):
            cu = (extract(uidx_v, k) // 128) * 128
            ci = (extract(iidx_v, k) // 128) * 128
            pltpu.async_copy(ut_hbm.at[pl.ds(0, H), pl.ds(cu, 128)],
                             u_pan.at[q, pl.ds(0, H)], sem_u.at[q])
            pltpu.async_copy(ut_hbm.at[pl.ds(H, H), pl.ds(cu, 128)],
                             u_pan.at[q, pl.ds(H, H)], sem_u.at[q])
            pltpu.async_copy(it_hbm.at[pl.ds(0, H), pl.ds(ci, 128)],
                             i_pan.at[q, pl.ds(0, H)], sem_i.at[q])
            pltpu.async_copy(it_hbm.at[pl.ds(H, H), pl.ds(ci, 128)],
                             i_pan.at[q, pl.ds(H, H)], sem_i.at[q])

        for q in range(Q):
            issue(q, q)

        def body(k0, carry):
            for q in range(Q):
                k = k0 * Q + q
                pltpu.make_async_copy(ut_hbm.at[:, pl.ds(0, 128)],
                                      u_pan.at[q], sem_u.at[q]).wait()
                pltpu.make_async_copy(it_hbm.at[:, pl.ds(0, 128)],
                                      i_pan.at[q], sem_i.at[q]).wait()
                ju = jnp.broadcast_to(extract(uidx_v, k) % 128, (LANES,))
                ji = jnp.broadcast_to(extract(iidx_v, k) % 128, (LANES,))
                acc = None
                for g in range(D // LANES):
                    dvec = g * LANES + iota
                    ug = plsc.load_gather(u_pan.at[q], [dvec, ju])
                    ig = plsc.load_gather(i_pan.at[q], [dvec, ji])
                    prod = ug * ig
                    acc = prod if acc is None else acc + prod
                s = jnp.broadcast_to(jnp.sum(acc), (LANES,))
                kn = k + Q
                pl.when(kn < BPW)(lambda: issue(kn, q))
                blk = (k // LANES) * LANES
                cur = out_v[pl.ds(blk, LANES)]
                out_v[pl.ds(blk, LANES)] = jnp.where(iota == (k % LANES),
                                                     s, cur)
            return carry

        lax.fori_loop(0, BPW // Q, body, 0)

        pltpu.sync_copy(out_v, out_hbm.at[pl.ds(base, BPW)])

    return _emb_dot(user.astype(jnp.int32), item.astype(jnp.int32), ut, it)


# speedup vs baseline: 1.0017x; 1.0017x over previous
import functools

import jax
import jax.numpy as jnp
from jax import lax
from jax.experimental import pallas as pl
from jax.experimental.pallas import tpu as pltpu
from jax.experimental.pallas import tpu_sc as plsc

NC = 2
NS = 16
NW = NC * NS
LANES = 16
Q = 4
P = 3


def kernel(user, item, user_table, item_table):
    B = user.shape[0]
    D = user_table.shape[1]
    BPW = B // NW

    ut = user_table.T
    it = item_table.T

    ui = user.astype(jnp.int32)
    ii = item.astype(jnp.int32)
    pos = lax.iota(jnp.int32, B)
    su, pu = lax.sort_key_val(ui, pos)
    si = jnp.take(ii, pu)

    pans = su >> 7
    prev = jnp.concatenate([jnp.full((1,), -1, jnp.int32), pans[:-1]])
    newp = ((pans != prev) | (pos % BPW == 0)).astype(jnp.int32)
    ordg = jnp.cumsum(newp) - 1
    ordb = jnp.repeat(ordg[:: BPW], BPW)
    slot = (ordg - ordb) % Q

    cm = (su << 3) | (slot << 1) | newp

    mesh = plsc.VectorSubcoreMesh(core_axis_name="c", subcore_axis_name="s")

    @functools.partial(
        pl.kernel,
        out_type=jax.ShapeDtypeStruct((B,), jnp.float32),
        mesh=mesh,
        scratch_types=[
            pltpu.VMEM((BPW,), jnp.int32),
            pltpu.VMEM((BPW,), jnp.int32),
            pltpu.VMEM((Q, D, 128), jnp.float32),
            pltpu.VMEM((Q, D, 128), jnp.float32),
            pltpu.VMEM((BPW,), jnp.float32),
            pltpu.SemaphoreType.DMA((Q,)),
            pltpu.SemaphoreType.DMA((Q,)),
        ],
        compiler_params=pltpu.CompilerParams(
            needs_layout_passes=False, use_tc_tiling_on_sc=True),
    )
    def _emb_dot(cm_hbm, si_hbm, ut_hbm, it_hbm, out_hbm,
                 cm_v, si_v, u_pan, i_pan, out_v, sem_u, sem_i):
        wid = lax.axis_index("s") * NC + lax.axis_index("c")
        base = wid * BPW

        pltpu.sync_copy(cm_hbm.at[pl.ds(base, BPW)], cm_v)
        pltpu.sync_copy(si_hbm.at[pl.ds(base, BPW)], si_v)

        iota = lax.iota(jnp.int32, LANES)
        zero = jnp.zeros((LANES,), jnp.int32)

        def extract(vref, k):
            vec = vref[pl.ds((k // LANES) * LANES, LANES)]
            return jnp.sum(jnp.where(iota == (k % LANES), vec, zero))

        def issue_user_at(k):
            m = extract(cm_v, k)

            @pl.when((m & 1) == 1)
            def _():
                sl = (m >> 1) & 3
                cu = pl.multiple_of((m >> 10) * 128, 128)
                pltpu.async_copy(ut_hbm.at[:, pl.ds(cu, 128)],
                                 u_pan.at[sl], sem_u.at[sl])

        def wait_user(sl):
            pltpu.make_async_copy(ut_hbm.at[:, pl.ds(0, 128)],
                                  u_pan.at[sl], sem_u.at[sl]).wait()

        def issue_item(k, q):
            ci = pl.multiple_of((extract(si_v, k) >> 7) * 128, 128)
            pltpu.async_copy(it_hbm.at[:, pl.ds(ci, 128)], i_pan.at[q],
                             sem_i.at[q])

        def wait_item(q):
            pltpu.make_async_copy(it_hbm.at[:, pl.ds(0, 128)],
                                  i_pan.at[q], sem_i.at[q]).wait()

        for kp in range(P):
            issue_user_at(kp)
        for q in range(Q):
            issue_item(q, q)

        def body(k0, carry):
            for q in range(Q):
                k = k0 * Q + q
                m = extract(cm_v, k)
                ri = extract(si_v, k)
                sl = (m >> 1) & 3

                @pl.when((m & 1) == 1)
                def _():
                    wait_user(sl)

                wait_item(q)

                ju = jnp.broadcast_to((m >> 3) & 127, (LANES,))
                ji = jnp.broadcast_to(ri & 127, (LANES,))
                slv = jnp.broadcast_to(sl, (LANES,))
                acc = None
                for g in range(D // LANES):
                    dvec = g * LANES + iota
                    ug = plsc.load_gather(u_pan, [slv, dvec, ju])
                    ig = plsc.load_gather(i_pan.at[q], [dvec, ji])
                    prod = ug * ig
                    acc = prod if acc is None else acc + prod
                s = jnp.broadcast_to(jnp.sum(acc), (LANES,))

                @pl.when(k + P < BPW)
                def _():
                    issue_user_at(k + P)

                kn = k + Q

                @pl.when(kn < BPW)
                def _():
                    issue_item(kn, q)

                blk = (k // LANES) * LANES
                cur = out_v[pl.ds(blk, LANES)]
                out_v[pl.ds(blk, LANES)] = jnp.where(iota == (k % LANES),
                                                     s, cur)
            return carry

        lax.fori_loop(0, BPW // Q, body, 0)

        pltpu.sync_copy(out_v, out_hbm.at[pl.ds(base, BPW)])

    res = _emb_dot(cm, si, ut, it)
    return jnp.zeros((B,), jnp.float32).at[pu].set(res)
